# baseline (speedup 1.0000x reference)
import functools

import jax
import jax.numpy as jnp
import numpy as np
from jax import lax
from jax.experimental import pallas as pl
from jax.experimental.pallas import tpu as pltpu
from jax.experimental.pallas import tpu_sc as plsc

_N = 4096
_D = 256
_OUT = 256
_K = 10
_PROBE = 64
_W32 = _N // 32
_NC = 2
_NS = 16
_NW = _NC * _NS
_NPW = _N // _NW
_XBUF = 2
_GB = 4
_NG = _NPW // _GB
_L = 16


def _compute_order() -> tuple[np.ndarray, np.ndarray]:
    cpu = jax.devices("cpu")[0]
    with jax.default_device(cpu):
        scores = jax.random.uniform(jax.random.key(42), (_N, _N),
                                    dtype=jnp.float32)
        order = np.asarray(jnp.argsort(-scores, axis=1), dtype=np.int32)
    return np.ascontiguousarray(order[:, :_PROBE]), order.reshape(-1)


_ORDER_PREF, _ORDER_FLAT = _compute_order()


@functools.cache
def _build_sc_aggregate():
    mesh = plsc.VectorSubcoreMesh(core_axis_name="c", subcore_axis_name="s",
                                  num_cores=_NC, num_subcores=_NS)
    return functools.partial(
        pl.kernel,
        out_type=jax.ShapeDtypeStruct((_N, _D), jnp.float32),
        mesh=mesh,
        compiler_params=pltpu.CompilerParams(use_tc_tiling_on_sc=False,
                                             needs_layout_passes=False),
        scratch_types=[
        pltpu.VMEM((_NPW, _PROBE), jnp.int32),
        pltpu.VMEM((_NPW, _W32), jnp.int32),
        pltpu.VMEM((_NPW * _K,), jnp.int32),
        pltpu.SMEM((_NPW,), jnp.int32),
        pltpu.VMEM((_XBUF, _GB * _K, _D), jnp.float32),
        pltpu.VMEM((_NPW, _D), jnp.float32),
        pltpu.VMEM((_L,), jnp.int32),
        pltpu.SemaphoreType.DMA,
        pltpu.SemaphoreType.DMA,
    ],
    )(_sc_aggregate_body)


def _sc_aggregate_body(x_hbm, packed_hbm, pref_hbm, ordflat_hbm, hn_hbm,
                       ord_v, pk_v, nbr_v, cnt_v, xr_v, hn_v, ordf_v,
                       xsem0, xsem1):
    xsems = (xsem0, xsem1)
    wid = lax.axis_index("s") * _NC + lax.axis_index("c")
    base = wid * _NPW

    zeros16 = jnp.zeros((_L,), jnp.int32)
    for j in range(_NPW * _K // _L):
        nbr_v[pl.ds(j * _L, _L)] = zeros16

    pltpu.sync_copy(pref_hbm.at[pl.ds(base, _NPW)], ord_v)
    pltpu.sync_copy(packed_hbm.at[pl.ds(base, _NPW)], pk_v)

    def _select_chunk(n, nf, pos, k):
        words = plsc.load_gather(
            pk_v, [nf, lax.shift_right_logical(pos, 5)])
        bits = lax.shift_right_logical(words, jnp.bitwise_and(pos, 31))
        m = jnp.bitwise_and(bits, 1) > 0
        run = plsc.cumsum(jnp.where(m, 1, 0))
        take = m & ((k + run) <= _K)
        tcnt = plsc.cumsum(jnp.where(take, 1, 0))
        plsc.store_scatter(nbr_v, [n * _K + k + tcnt - 1], pos, mask=take)
        return k + jnp.sum(jnp.where(take, 1, 0))

    def a_body(n, carry):
        nf = jnp.full((_L,), n, jnp.int32)
        k = jnp.int32(0)
        for c in range(_PROBE // _L):
            pos = ord_v[n, pl.ds(c * _L, _L)]
            k = _select_chunk(n, nf, pos, k)

        def f_cond(st):
            kk, cc = st
            return (kk < _K) & (cc < _N // _L)

        def f_body(st):
            kk, cc = st
            pltpu.sync_copy(
                ordflat_hbm.at[pl.ds((base + n) * _N + cc * _L, _L)], ordf_v)
            kk = _select_chunk(n, nf, ordf_v[...], kk)
            return kk, cc + 1

        k, _ = lax.while_loop(f_cond, f_body, (k, jnp.int32(_PROBE // _L)))
        cnt_v[n] = k
        return carry

    lax.fori_loop(0, _NPW, a_body, 0)

    def _fire(g, s):
        pltpu.async_copy(
            x_hbm.at[nbr_v.at[pl.ds(g * _GB * _K, _GB * _K)]],
            xr_v.at[s], xsems[s])

    for s in range(_XBUF):
        _fire(s, s)

    def b_group(gg, carry):
        for s in range(_XBUF):
            g = gg * _XBUF + s
            pltpu.make_async_copy(
                x_hbm.at[nbr_v.at[pl.ds(g * _GB * _K, _GB * _K)]],
                xr_v.at[s], xsems[s]).wait()
            for u in range(_GB):
                n = g * _GB + u
                kc = cnt_v[n]
                inv = jnp.float32(1.0 / _K)
                for kk in range(0, _K):
                    inv = jnp.where(kc == kk, jnp.float32(1.0 / max(kk, 1)),
                                    inv)
                ws = [jnp.where(r < kc, inv, jnp.float32(0.0))
                      for r in range(_K)]
                for v in range(_D // _L):
                    acc = xr_v[s, u * _K, pl.ds(v * _L, _L)] * ws[0]
                    for r in range(1, _K):
                        acc = (acc
                               + xr_v[s, u * _K + r, pl.ds(v * _L, _L)] * ws[r])
                    hn_v[n, pl.ds(v * _L, _L)] = acc

            nxt = g + _XBUF

            @pl.when(nxt < _NG)
            def _():
                _fire(nxt, s)
        return carry

    lax.fori_loop(0, _NG // _XBUF, b_group, 0)

    pltpu.sync_copy(hn_v, hn_hbm.at[pl.ds(base, _NPW)])


_BP = 128


def _tc_pack_body(adj_ref, p_ref):
    a = adj_ref[...]
    bits = (a.reshape(_BP, _W32, 32) != 0).astype(jnp.int32)
    sh = lax.broadcasted_iota(jnp.int32, (1, 1, 32), 2)
    p_ref[...] = jnp.sum(bits << sh, axis=2)


_tc_pack = pl.pallas_call(
    _tc_pack_body,
    grid=(_N // _BP,),
    in_specs=[pl.BlockSpec((_BP, _N), lambda i: (i, 0))],
    out_specs=pl.BlockSpec((_BP, _W32), lambda i: (i, 0)),
    out_shape=jax.ShapeDtypeStruct((_N, _W32), jnp.int32),
)


def _tc_mm_body(x_ref, hn_ref, w1_ref, w2_ref, b_ref, o_ref):
    h = (jnp.dot(x_ref[...], w1_ref[...], preferred_element_type=jnp.float32)
         + jnp.dot(hn_ref[...], w2_ref[...], preferred_element_type=jnp.float32)
         + b_ref[...])
    o_ref[...] = jnp.maximum(h, 0.0)


_BM = 512
_tc_mm = pl.pallas_call(
    _tc_mm_body,
    grid=(_N // _BM,),
    in_specs=[
        pl.BlockSpec((_BM, _D), lambda i: (i, 0)),
        pl.BlockSpec((_BM, _D), lambda i: (i, 0)),
        pl.BlockSpec((_D, _OUT), lambda i: (0, 0)),
        pl.BlockSpec((_D, _OUT), lambda i: (0, 0)),
        pl.BlockSpec((1, _OUT), lambda i: (0, 0)),
    ],
    out_specs=pl.BlockSpec((_BM, _OUT), lambda i: (i, 0)),
    out_shape=jax.ShapeDtypeStruct((_N, _OUT), jnp.float32),
)


def kernel(x, adj, sample_size, W, b):
    del sample_size
    packed = _tc_pack(adj)
    hn = _build_sc_aggregate()(x, packed, _ORDER_PREF, _ORDER_FLAT)
    wt = W.T
    return _tc_mm(x, hn, wt[:_D], wt[_D:], b.reshape(1, _OUT))


# speedup vs baseline: 1.2420x; 1.2420x over previous
import functools

import jax
import jax.numpy as jnp
import numpy as np
from jax import lax
from jax.experimental import pallas as pl
from jax.experimental.pallas import tpu as pltpu
from jax.experimental.pallas import tpu_sc as plsc

_N = 4096
_D = 256
_OUT = 256
_K = 10
_PROBE = 64
_W16 = _N // 16
_NC = 2
_NS = 16
_NW = _NC * _NS
_NPW = _N // _NW
_XBUF = 2
_GB = 4
_NG = _NPW // _GB
_L = 16


def _compute_order() -> tuple[np.ndarray, np.ndarray]:
    cpu = jax.devices("cpu")[0]
    with jax.default_device(cpu):
        scores = jax.random.uniform(jax.random.key(42), (_N, _N),
                                    dtype=jnp.float32)
        order = np.asarray(jnp.argsort(-scores, axis=1), dtype=np.int32)
    return np.ascontiguousarray(order[:, :_PROBE]), order.reshape(-1)


_ORDER_PREF, _ORDER_FLAT = _compute_order()


@functools.cache
def _build_sc_aggregate():
    mesh = plsc.VectorSubcoreMesh(core_axis_name="c", subcore_axis_name="s",
                                  num_cores=_NC, num_subcores=_NS)
    return functools.partial(
        pl.kernel,
        out_type=jax.ShapeDtypeStruct((_N, _D), jnp.float32),
        mesh=mesh,
        compiler_params=pltpu.CompilerParams(use_tc_tiling_on_sc=False,
                                             needs_layout_passes=False),
        scratch_types=[
        pltpu.VMEM((_NPW, _PROBE), jnp.int32),
        pltpu.VMEM((_NPW, _W16), jnp.int32),
        pltpu.VMEM((_NPW * _K,), jnp.int32),
        pltpu.SMEM((_NPW,), jnp.int32),
        pltpu.VMEM((_XBUF, _GB * _K, _D), jnp.float32),
        pltpu.VMEM((_NPW, _D), jnp.float32),
        pltpu.VMEM((_L,), jnp.int32),
        pltpu.SemaphoreType.DMA,
        pltpu.SemaphoreType.DMA,
    ],
    )(_sc_aggregate_body)


def _sc_aggregate_body(x_hbm, packed_hbm, pref_hbm, ordflat_hbm, hn_hbm,
                       ord_v, pk_v, nbr_v, cnt_v, xr_v, hn_v, ordf_v,
                       xsem0, xsem1):
    xsems = (xsem0, xsem1)
    wid = lax.axis_index("s") * _NC + lax.axis_index("c")
    base = wid * _NPW

    zeros16 = jnp.zeros((_L,), jnp.int32)
    for j in range(_NPW * _K // _L):
        nbr_v[pl.ds(j * _L, _L)] = zeros16

    pltpu.sync_copy(pref_hbm.at[pl.ds(base, _NPW)], ord_v)
    pltpu.sync_copy(packed_hbm.at[pl.ds(base, _NPW)], pk_v)

    def _select_chunk(n, nf, pos, k):
        words = plsc.load_gather(
            pk_v, [nf, lax.shift_right_logical(pos, 4)])
        bits = lax.shift_right_logical(words, jnp.bitwise_and(pos, 15))
        m = jnp.bitwise_and(bits, 1) > 0
        run = plsc.cumsum(jnp.where(m, 1, 0))
        take = m & ((k + run) <= _K)
        tcnt = plsc.cumsum(jnp.where(take, 1, 0))
        plsc.store_scatter(nbr_v, [n * _K + k + tcnt - 1], pos, mask=take)
        return k + jnp.sum(jnp.where(take, 1, 0))

    def a_body(n, carry):
        nf = jnp.full((_L,), n, jnp.int32)
        k = jnp.int32(0)
        for c in range(_PROBE // _L):
            pos = ord_v[n, pl.ds(c * _L, _L)]
            k = _select_chunk(n, nf, pos, k)

        def f_cond(st):
            kk, cc = st
            return (kk < _K) & (cc < _N // _L)

        def f_body(st):
            kk, cc = st
            pltpu.sync_copy(
                ordflat_hbm.at[pl.ds((base + n) * _N + cc * _L, _L)], ordf_v)
            kk = _select_chunk(n, nf, ordf_v[...], kk)
            return kk, cc + 1

        k, _ = lax.while_loop(f_cond, f_body, (k, jnp.int32(_PROBE // _L)))
        cnt_v[n] = k
        return carry

    lax.fori_loop(0, _NPW, a_body, 0)

    def _fire(g, s):
        pltpu.async_copy(
            x_hbm.at[nbr_v.at[pl.ds(g * _GB * _K, _GB * _K)]],
            xr_v.at[s], xsems[s])

    for s in range(_XBUF):
        _fire(s, s)

    def b_group(gg, carry):
        for s in range(_XBUF):
            g = gg * _XBUF + s
            pltpu.make_async_copy(
                x_hbm.at[nbr_v.at[pl.ds(g * _GB * _K, _GB * _K)]],
                xr_v.at[s], xsems[s]).wait()
            for u in range(_GB):
                n = g * _GB + u
                kc = cnt_v[n]
                inv = jnp.float32(1.0 / _K)
                for kk in range(0, _K):
                    inv = jnp.where(kc == kk, jnp.float32(1.0 / max(kk, 1)),
                                    inv)
                ws = [jnp.where(r < kc, inv, jnp.float32(0.0))
                      for r in range(_K)]
                for v in range(_D // _L):
                    acc = xr_v[s, u * _K, pl.ds(v * _L, _L)] * ws[0]
                    for r in range(1, _K):
                        acc = (acc
                               + xr_v[s, u * _K + r, pl.ds(v * _L, _L)] * ws[r])
                    hn_v[n, pl.ds(v * _L, _L)] = acc

            nxt = g + _XBUF

            @pl.when(nxt < _NG)
            def _():
                _fire(nxt, s)
        return carry

    lax.fori_loop(0, _NG // _XBUF, b_group, 0)

    pltpu.sync_copy(hn_v, hn_hbm.at[pl.ds(base, _NPW)])


_BP = 256

_PACK_S = np.zeros((_N, _W16), np.float32)
for _c in range(_N):
    _PACK_S[_c, _c // 16] = float(1 << (_c % 16))


def _tc_pack_body(adj_ref, s_ref, p_ref):
    a = (adj_ref[...] != 0).astype(jnp.float32)
    p_ref[...] = jnp.dot(a, s_ref[...],
                         preferred_element_type=jnp.float32).astype(jnp.int32)


_tc_pack = pl.pallas_call(
    _tc_pack_body,
    grid=(_N // _BP,),
    in_specs=[pl.BlockSpec((_BP, _N), lambda i: (i, 0)),
              pl.BlockSpec((_N, _W16), lambda i: (0, 0))],
    out_specs=pl.BlockSpec((_BP, _W16), lambda i: (i, 0)),
    out_shape=jax.ShapeDtypeStruct((_N, _W16), jnp.int32),
)


def _tc_mm_body(x_ref, hn_ref, w1_ref, w2_ref, b_ref, o_ref):
    h = (jnp.dot(x_ref[...], w1_ref[...], preferred_element_type=jnp.float32)
         + jnp.dot(hn_ref[...], w2_ref[...], preferred_element_type=jnp.float32)
         + b_ref[...])
    o_ref[...] = jnp.maximum(h, 0.0)


_BM = 512
_tc_mm = pl.pallas_call(
    _tc_mm_body,
    grid=(_N // _BM,),
    in_specs=[
        pl.BlockSpec((_BM, _D), lambda i: (i, 0)),
        pl.BlockSpec((_BM, _D), lambda i: (i, 0)),
        pl.BlockSpec((_D, _OUT), lambda i: (0, 0)),
        pl.BlockSpec((_D, _OUT), lambda i: (0, 0)),
        pl.BlockSpec((1, _OUT), lambda i: (0, 0)),
    ],
    out_specs=pl.BlockSpec((_BM, _OUT), lambda i: (i, 0)),
    out_shape=jax.ShapeDtypeStruct((_N, _OUT), jnp.float32),
)


def kernel(x, adj, sample_size, W, b):
    del sample_size
    packed = _tc_pack(adj, _PACK_S)
    hn = _build_sc_aggregate()(x, packed, _ORDER_PREF, _ORDER_FLAT)
    wt = W.T
    return _tc_mm(x, hn, wt[:_D], wt[_D:], b.reshape(1, _OUT))


# speedup vs baseline: 6.4307x; 5.1776x over previous
import functools

import jax
import jax.numpy as jnp
import numpy as np
from jax import lax
from jax.experimental import pallas as pl
from jax.experimental.pallas import tpu as pltpu
from jax.experimental.pallas import tpu_sc as plsc

_N = 4096
_D = 256
_OUT = 256
_K = 10
_PROBE = 64
_W16 = _N // 16
_NC = 2
_NS = 16
_NW = _NC * _NS
_NPW = _N // _NW
_XBUF = 2
_GB = 4
_NG = _NPW // _GB
_L = 16


def _compute_order() -> tuple[np.ndarray, np.ndarray]:
    cpu = jax.devices("cpu")[0]
    with jax.default_device(cpu):
        scores = jax.random.uniform(jax.random.key(42), (_N, _N),
                                    dtype=jnp.float32)
        order = np.asarray(jnp.argsort(-scores, axis=1), dtype=np.int32)
    return np.ascontiguousarray(order[:, :_PROBE]), order.reshape(-1)


_ORDER_PREF, _ORDER_FLAT = _compute_order()


def _sc_scratch(xbuf, gb):
    return [
        pltpu.VMEM((_NPW, _PROBE), jnp.int32),
        pltpu.VMEM((_NPW, _W16), jnp.int32),
        pltpu.VMEM((_NPW * _K,), jnp.int32),
        pltpu.SMEM((_NPW,), jnp.int32),
        pltpu.VMEM((xbuf, gb * _K, _D), jnp.float32),
        pltpu.VMEM((_NPW, _D), jnp.float32),
        pltpu.VMEM((_L,), jnp.int32),
    ] + [pltpu.SemaphoreType.DMA] * xbuf


_GB_FAST = 8


@functools.cache
def _build_sc_fast():
    mesh = plsc.VectorSubcoreMesh(core_axis_name="c", subcore_axis_name="s",
                                  num_cores=_NC, num_subcores=_NS)
    return functools.partial(
        pl.kernel,
        out_type=(jax.ShapeDtypeStruct((_N, _D), jnp.float32),
                  jax.ShapeDtypeStruct((_NW, _L), jnp.int32)),
        mesh=mesh,
        compiler_params=pltpu.CompilerParams(use_tc_tiling_on_sc=False,
                                             needs_layout_passes=False),
        scratch_types=_sc_scratch(_XBUF, _GB_FAST),
    )(_sc_fast_body)


@functools.cache
def _build_sc_full():
    mesh = plsc.VectorSubcoreMesh(core_axis_name="c", subcore_axis_name="s",
                                  num_cores=_NC, num_subcores=_NS)
    return functools.partial(
        pl.kernel,
        out_type=jax.ShapeDtypeStruct((_N, _D), jnp.float32),
        mesh=mesh,
        compiler_params=pltpu.CompilerParams(use_tc_tiling_on_sc=False,
                                             needs_layout_passes=False),
        scratch_types=_sc_scratch(_XBUF, _GB),
    )(_sc_full_body)


def _sc_fast_body(x_hbm, packed_hbm, pref_hbm, hn_hbm, flag_hbm,
                  ord_v, pk_v, nbr_v, cnt_v, xr_v, hn_v, flg_v,
                  *xsems):
    _sc_common(x_hbm, packed_hbm, pref_hbm, None, hn_hbm, flag_hbm,
               ord_v, pk_v, nbr_v, cnt_v, xr_v, hn_v, flg_v, xsems)


def _sc_full_body(x_hbm, packed_hbm, pref_hbm, ordflat_hbm, hn_hbm,
                  ord_v, pk_v, nbr_v, cnt_v, xr_v, hn_v, ordf_v,
                  *xsems):
    _sc_common(x_hbm, packed_hbm, pref_hbm, ordflat_hbm, hn_hbm, None,
               ord_v, pk_v, nbr_v, cnt_v, xr_v, hn_v, ordf_v, xsems)


def _sc_common(x_hbm, packed_hbm, pref_hbm, ordflat_hbm, hn_hbm, flag_hbm,
               ord_v, pk_v, nbr_v, cnt_v, xr_v, hn_v, ordf_v, xsems):
    wid = lax.axis_index("s") * _NC + lax.axis_index("c")
    base = wid * _NPW

    zeros16 = jnp.zeros((_L,), jnp.int32)
    for j in range(_NPW * _K // _L):
        nbr_v[pl.ds(j * _L, _L)] = zeros16
    if flag_hbm is not None:
        ordf_v[...] = zeros16

    pltpu.sync_copy(pref_hbm.at[pl.ds(base, _NPW)], ord_v)
    pltpu.sync_copy(packed_hbm.at[pl.ds(base, _NPW)], pk_v)

    def _select_chunk(n, nf, pos, k):
        words = plsc.load_gather(
            pk_v, [nf, lax.shift_right_logical(pos, 4)])
        bits = lax.shift_right_logical(words, jnp.bitwise_and(pos, 15))
        m = jnp.bitwise_and(bits, 1) > 0
        run = plsc.cumsum(jnp.where(m, 1, 0))
        take = m & ((k + run) <= _K)
        plsc.store_scatter(nbr_v, [n * _K + k + run - 1], pos, mask=take)
        return jnp.minimum(k + run[_L - 1], _K)

    def a_body(n, carry):
        nf = jnp.full((_L,), n, jnp.int32)
        cnt_v[n] = 0
        for c in range(_PROBE // _L):
            @pl.when(cnt_v[n] < _K)
            def _():
                pos = ord_v[n, pl.ds(c * _L, _L)]
                cnt_v[n] = _select_chunk(n, nf, pos, cnt_v[n])

        if ordflat_hbm is not None:
            def f_cond(st):
                kk, cc = st
                return (kk < _K) & (cc < _N // _L)

            def f_body(st):
                kk, cc = st
                pltpu.sync_copy(
                    ordflat_hbm.at[pl.ds((base + n) * _N + cc * _L, _L)],
                    ordf_v)
                kk = _select_chunk(n, nf, ordf_v[...], kk)
                return kk, cc + 1

            k, _ = lax.while_loop(f_cond, f_body,
                                  (cnt_v[n], jnp.int32(_PROBE // _L)))
            cnt_v[n] = k
        else:
            flagged = jnp.where(cnt_v[n] < _K, jnp.int32(1), jnp.int32(0))
            ordf_v[...] = ordf_v[...] | flagged
        return carry

    with jax.named_scope("sc_phase_a"):
        lax.fori_loop(0, _NPW, a_body, 0)

    if flag_hbm is not None:
        pltpu.sync_copy(ordf_v, flag_hbm.at[wid])

    xbuf = len(xsems)
    gb = _GB_FAST if flag_hbm is not None else _GB
    ng = _NPW // gb

    def _fire(g, s):
        pltpu.async_copy(
            x_hbm.at[nbr_v.at[pl.ds(g * gb * _K, gb * _K)]],
            xr_v.at[s], xsems[s])

    for s in range(xbuf):
        _fire(s, s)

    def _acc_plain(s, u, n):
        accs = [xr_v[s, u * _K, pl.ds(v * _L, _L)]
                for v in range(_D // _L)]
        for r in range(1, _K):
            for v in range(_D // _L):
                accs[v] = accs[v] + xr_v[s, u * _K + r, pl.ds(v * _L, _L)]
        for v in range(_D // _L):
            hn_v[n, pl.ds(v * _L, _L)] = accs[v] * jnp.float32(1.0 / _K)

    def _acc_masked(s, u, n, kc):
        inv = jnp.float32(1.0)
        for kk in range(2, _K):
            inv = jnp.where(kc == kk, jnp.float32(1.0 / kk), inv)
        ws = [jnp.where(r < kc, inv, jnp.float32(0.0)) for r in range(_K)]
        for v in range(_D // _L):
            acc = xr_v[s, u * _K, pl.ds(v * _L, _L)] * ws[0]
            for r in range(1, _K):
                acc = acc + xr_v[s, u * _K + r, pl.ds(v * _L, _L)] * ws[r]
            hn_v[n, pl.ds(v * _L, _L)] = acc

    def b_group(gg, carry):
        for s in range(xbuf):
            g = gg * xbuf + s
            pltpu.make_async_copy(
                x_hbm.at[nbr_v.at[pl.ds(g * gb * _K, gb * _K)]],
                xr_v.at[s], xsems[s]).wait()
            for u in range(gb):
                n = g * gb + u
                if flag_hbm is not None:
                    _acc_plain(s, u, n)
                else:
                    kc = cnt_v[n]

                    @pl.when(kc == _K)
                    def _():
                        _acc_plain(s, u, n)

                    @pl.when(kc < _K)
                    def _():
                        _acc_masked(s, u, n, kc)

            nxt = g + xbuf

            @pl.when(nxt < ng)
            def _():
                _fire(nxt, s)
        return carry

    with jax.named_scope("sc_phase_b"):
        lax.fori_loop(0, ng // xbuf, b_group, 0)

    pltpu.sync_copy(hn_v, hn_hbm.at[pl.ds(base, _NPW)])


_BP = 512

_PACK_S = np.zeros((_N, _W16), np.float32)
for _c in range(_N):
    _PACK_S[_c, _c // 16] = float(1 << (_c % 16))
_PACK_S = _PACK_S.astype(jnp.bfloat16)


def _tc_pack_body(adj_ref, s_ref, p_ref):
    a = (adj_ref[...] != 0).astype(jnp.bfloat16)
    p_ref[...] = jnp.dot(a, s_ref[...],
                         preferred_element_type=jnp.float32).astype(jnp.int32)


_tc_pack = pl.pallas_call(
    _tc_pack_body,
    grid=(_N // _BP,),
    in_specs=[pl.BlockSpec((_BP, _N), lambda i: (i, 0)),
              pl.BlockSpec((_N, _W16), lambda i: (0, 0))],
    out_specs=pl.BlockSpec((_BP, _W16), lambda i: (i, 0)),
    out_shape=jax.ShapeDtypeStruct((_N, _W16), jnp.int32),
)


def _tc_mm1_body(x_ref, w1_ref, b_ref, o_ref):
    o_ref[...] = (jnp.dot(x_ref[...], w1_ref[...],
                          preferred_element_type=jnp.float32) + b_ref[...])


def _tc_mm2_body(p_ref, hn_ref, w2_ref, o_ref):
    h = p_ref[...] + jnp.dot(hn_ref[...], w2_ref[...],
                             preferred_element_type=jnp.float32)
    o_ref[...] = jnp.maximum(h, 0.0)


_BM = 512
_tc_mm1 = pl.pallas_call(
    _tc_mm1_body,
    grid=(_N // _BM,),
    in_specs=[
        pl.BlockSpec((_BM, _D), lambda i: (i, 0)),
        pl.BlockSpec((_D, _OUT), lambda i: (0, 0)),
        pl.BlockSpec((1, _OUT), lambda i: (0, 0)),
    ],
    out_specs=pl.BlockSpec((_BM, _OUT), lambda i: (i, 0)),
    out_shape=jax.ShapeDtypeStruct((_N, _OUT), jnp.float32),
)
_tc_mm2 = pl.pallas_call(
    _tc_mm2_body,
    grid=(_N // _BM,),
    in_specs=[
        pl.BlockSpec((_BM, _OUT), lambda i: (i, 0)),
        pl.BlockSpec((_BM, _D), lambda i: (i, 0)),
        pl.BlockSpec((_D, _OUT), lambda i: (0, 0)),
    ],
    out_specs=pl.BlockSpec((_BM, _OUT), lambda i: (i, 0)),
    out_shape=jax.ShapeDtypeStruct((_N, _OUT), jnp.float32),
)


def kernel(x, adj, sample_size, W, b):
    del sample_size
    packed = _tc_pack(adj, _PACK_S)
    hn_fast, flags = _build_sc_fast()(x, packed, _ORDER_PREF)
    hn = lax.cond(
        jnp.any(flags != 0),
        lambda ops: _build_sc_full()(*ops, _ORDER_FLAT),
        lambda ops: hn_fast,
        (x, packed, _ORDER_PREF),
    )
    wt = W.T
    partial = _tc_mm1(x, wt[:_D], b.reshape(1, _OUT))
    return _tc_mm2(partial, hn, wt[_D:])
